# baseline (speedup 1.0000x reference)
import functools

import jax
import jax.numpy as jnp
from jax import lax
from jax.experimental import pallas as pl
from jax.experimental.pallas import tpu as pltpu
from jax.experimental.pallas import tpu_sc as plsc

BATCH = 4096
SEQ = 200
D = 64
NUM_CLASSES = 5

NUM_CORES = 2
NUM_SUBCORES = 16
NUM_WORKERS = NUM_CORES * NUM_SUBCORES
B_PER_W = BATCH // NUM_WORKERS
SEQ_PAD = 208
CHUNKS = ((0, 128), (128, 72))
LANES = 16
DV = D // LANES


def _sc_pooled_sums(x, table):
  mesh = plsc.VectorSubcoreMesh(core_axis_name="c", subcore_axis_name="s")

  @functools.partial(
      pl.kernel,
      mesh=mesh,
      compiler_params=pltpu.CompilerParams(use_tc_tiling_on_sc=False),
      out_type=jax.ShapeDtypeStruct((BATCH, D), jnp.float32),
      scratch_types=[
          pltpu.VMEM((B_PER_W, SEQ_PAD), jnp.int32),
          pltpu.VMEM((SEQ, D), jnp.float32),
          pltpu.VMEM((B_PER_W, D), jnp.float32),
          pltpu.SemaphoreType.DMA,
      ],
  )
  def sc_kernel(x_hbm, table_hbm, out_hbm, idx_v, rows_v, acc_v, sem):
    wid = lax.axis_index("s") * NUM_CORES + lax.axis_index("c")
    base = wid * B_PER_W
    pltpu.sync_copy(
        x_hbm.at[pl.ds(base, B_PER_W)],
        idx_v.at[:, pl.ds(0, SEQ)],
    )

    def row_body(b, _):
      copies = []
      for off, ln in CHUNKS:
        copies.append(
            pltpu.async_copy(
                table_hbm.at[idx_v.at[b, pl.ds(off, ln)]],
                rows_v.at[pl.ds(off, ln)],
                sem,
            )
        )
      for c in copies:
        c.wait()

      def red_body(r, carry):
        out = []
        for k in range(DV):
          a = carry[k]
          a = a + rows_v[r, pl.ds(k * LANES, LANES)]
          a = a + rows_v[r + 1, pl.ds(k * LANES, LANES)]
          out.append(a)
        return tuple(out)

      zeros = tuple(jnp.zeros((LANES,), jnp.float32) for _ in range(DV))
      acc = lax.fori_loop(0, SEQ // 2, lambda r, c: red_body(2 * r, c), zeros)
      for k in range(DV):
        acc_v[b, pl.ds(k * LANES, LANES)] = acc[k]
      return 0

    lax.fori_loop(0, B_PER_W, row_body, 0)
    pltpu.sync_copy(acc_v, out_hbm.at[pl.ds(base, B_PER_W)])

  return sc_kernel(x, table)


def _tc_epilogue(sums, x, t0, W, b):
  def tc_kernel(sums_ref, x_ref, t0_ref, w_ref, b_ref, out_ref):
    n0 = jnp.sum((x_ref[...] == 0).astype(jnp.float32), axis=1, keepdims=True)
    mean = (sums_ref[...] - n0 * t0_ref[...]) * (1.0 / SEQ)
    out_ref[...] = (
        jnp.dot(mean, w_ref[...].T, preferred_element_type=jnp.float32)
        + b_ref[...]
    )

  return pl.pallas_call(
      tc_kernel,
      out_shape=jax.ShapeDtypeStruct((BATCH, NUM_CLASSES), jnp.float32),
  )(sums, x, t0, W, b)


def kernel(x, table, W, b):
  sums = _sc_pooled_sums(x, table)
  t0 = lax.slice(table, (0, 0), (1, D))
  return _tc_epilogue(sums, x, t0, W, b.reshape(1, NUM_CLASSES))


# speedup vs baseline: 1.1346x; 1.1346x over previous
import functools

import jax
import jax.numpy as jnp
from jax import lax
from jax.experimental import pallas as pl
from jax.experimental.pallas import tpu as pltpu
from jax.experimental.pallas import tpu_sc as plsc

BATCH = 4096
SEQ = 200
D = 64
NUM_CLASSES = 5

NUM_CORES = 2
NUM_SUBCORES = 16
NUM_WORKERS = NUM_CORES * NUM_SUBCORES
B_PER_W = BATCH // NUM_WORKERS
SEQ_PAD = 208
CHUNKS = ((0, 128), (128, 72))
LANES = 16
DV = D // LANES


def _sc_pooled_sums(x, table):
  mesh = plsc.VectorSubcoreMesh(core_axis_name="c", subcore_axis_name="s")

  @functools.partial(
      pl.kernel,
      mesh=mesh,
      compiler_params=pltpu.CompilerParams(use_tc_tiling_on_sc=False),
      out_type=jax.ShapeDtypeStruct((BATCH, D), jnp.float32),
      scratch_types=[
          pltpu.VMEM((B_PER_W, SEQ_PAD), jnp.int32),
          pltpu.VMEM((2, SEQ, D), jnp.float32),
          pltpu.VMEM((B_PER_W, D), jnp.float32),
          pltpu.SemaphoreType.DMA,
          pltpu.SemaphoreType.DMA,
      ],
  )
  def sc_kernel(x_hbm, table_hbm, out_hbm, idx_v, rows_v, acc_v, sem0, sem1):
    wid = lax.axis_index("s") * NUM_CORES + lax.axis_index("c")
    base = wid * B_PER_W
    sems = (sem0, sem1)
    pltpu.sync_copy(
        x_hbm.at[pl.ds(base, B_PER_W)],
        idx_v.at[:, pl.ds(0, SEQ)],
    )

    def issue(b, buf):
      for off, ln in CHUNKS:
        pltpu.async_copy(
            table_hbm.at[idx_v.at[b, pl.ds(off, ln)]],
            rows_v.at[buf, pl.ds(off, ln)],
            sems[buf],
        )

    def wait(b, buf):
      for off, ln in CHUNKS:
        pltpu.make_async_copy(
            table_hbm.at[idx_v.at[b, pl.ds(off, ln)]],
            rows_v.at[buf, pl.ds(off, ln)],
            sems[buf],
        ).wait()

    def reduce_into(b, buf):
      def red_body(r, carry):
        out = []
        for k in range(DV):
          a = carry[k]
          a = a + rows_v[buf, r, pl.ds(k * LANES, LANES)]
          a = a + rows_v[buf, r + 1, pl.ds(k * LANES, LANES)]
          out.append(a)
        return tuple(out)

      zeros = tuple(jnp.zeros((LANES,), jnp.float32) for _ in range(DV))
      acc = lax.fori_loop(0, SEQ // 2, lambda r, c: red_body(2 * r, c), zeros)
      for k in range(DV):
        acc_v[b, pl.ds(k * LANES, LANES)] = acc[k]

    issue(0, 0)

    def pair_body(p, _):
      b0 = 2 * p
      issue(b0 + 1, 1)
      wait(b0, 0)
      reduce_into(b0, 0)

      @pl.when(p < B_PER_W // 2 - 1)
      def _():
        issue(b0 + 2, 0)

      wait(b0 + 1, 1)
      reduce_into(b0 + 1, 1)
      return 0

    lax.fori_loop(0, B_PER_W // 2, pair_body, 0)
    pltpu.sync_copy(acc_v, out_hbm.at[pl.ds(base, B_PER_W)])

  return sc_kernel(x, table)


def _tc_epilogue(sums, x, t0, W, b):
  def tc_kernel(sums_ref, x_ref, t0_ref, w_ref, b_ref, out_ref):
    n0 = jnp.sum((x_ref[...] == 0).astype(jnp.float32), axis=1, keepdims=True)
    mean = (sums_ref[...] - n0 * t0_ref[...]) * (1.0 / SEQ)
    out_ref[...] = (
        jnp.dot(mean, w_ref[...].T, preferred_element_type=jnp.float32)
        + b_ref[...]
    )

  return pl.pallas_call(
      tc_kernel,
      out_shape=jax.ShapeDtypeStruct((BATCH, NUM_CLASSES), jnp.float32),
  )(sums, x, t0, W, b)


def kernel(x, table, W, b):
  sums = _sc_pooled_sums(x, table)
  t0 = lax.slice(table, (0, 0), (1, D))
  return _tc_epilogue(sums, x, t0, W, b.reshape(1, NUM_CLASSES))
